# baseline (speedup 1.0000x reference)
import functools

import jax
import jax.numpy as jnp
import numpy as np
from jax import lax
from jax.experimental import pallas as pl
from jax.experimental.pallas import tpu as pltpu
from jax.experimental.pallas import tpu_sc as plsc

_DIM = 64
_NUM_REL = 500
_REL_PAD = 512

_NC = 2
_NS = 16
_NW = _NC * _NS
_GCHUNK = 128


@functools.lru_cache(maxsize=None)
def _gather_kernel(total_rows: int, dim: int):
    rows_per_w = total_rows // _NW
    n_chunks = rows_per_w // _GCHUNK
    mesh = plsc.VectorSubcoreMesh(core_axis_name="c", subcore_axis_name="s")

    @functools.partial(
        pl.kernel,
        mesh=mesh,
        out_type=jax.ShapeDtypeStruct((total_rows, dim), jnp.float32),
        scratch_types=[
            pltpu.VMEM((rows_per_w,), jnp.int32),
            pltpu.VMEM((rows_per_w, dim), jnp.float32),
            pltpu.SemaphoreType.DMA,
        ],
        compiler_params=pltpu.CompilerParams(use_tc_tiling_on_sc=False),
    )
    def gather(idx_hbm, table_hbm, out_hbm, idx_v, rows_v, sem):
        wid = lax.axis_index("s") * _NC + lax.axis_index("c")
        base = wid * rows_per_w
        pltpu.sync_copy(idx_hbm.at[pl.ds(base, rows_per_w)], idx_v)
        copies = [
            pltpu.async_copy(
                table_hbm.at[idx_v.at[pl.ds(j * _GCHUNK, _GCHUNK)]],
                rows_v.at[pl.ds(j * _GCHUNK, _GCHUNK), :],
                sem,
            )
            for j in range(n_chunks)
        ]
        for cp in copies:
            cp.wait()
        pltpu.sync_copy(rows_v, out_hbm.at[pl.ds(base, rows_per_w)])

    return gather


_TR_E = 8192
_TR_N = 122
_TR_NBUF = 4
_TR_TAIL = 576
_TR_GROUPS = 30


def _stream_tr_body(eye_ref, tail_ref, in_hbm, out_hbm, *rest):
    bins = rest[0:_TR_NBUF]
    bouts = rest[_TR_NBUF : 2 * _TR_NBUF]
    sins = rest[2 * _TR_NBUF : 3 * _TR_NBUF]
    souts = rest[3 * _TR_NBUF : 4 * _TR_NBUF]
    stail = rest[4 * _TR_NBUF]

    def start_in(i, b):
        pltpu.make_async_copy(
            in_hbm.at[:, pl.ds(i * _TR_E, _TR_E)], bins[b], sins[b]
        ).start()

    def wait_in(b):
        pltpu.make_async_copy(
            in_hbm.at[:, pl.ds(0, _TR_E)], bins[b], sins[b]
        ).wait()

    def start_out(i, b):
        pltpu.make_async_copy(
            bouts[b], out_hbm.at[pl.ds(i * _TR_E, _TR_E), :], souts[b]
        ).start()

    def wait_out(b):
        pltpu.make_async_copy(
            bouts[b], out_hbm.at[pl.ds(0, _TR_E), :], souts[b]
        ).wait()

    for b in range(_TR_NBUF):
        start_in(b, b)

    def group(j, carry):
        for b in range(_TR_NBUF):
            i = j * _TR_NBUF + b
            wait_in(b)

            @pl.when(j > 0)
            def _():
                wait_out(b)

            bouts[b][...] = bins[b][...].T

            @pl.when(j < _TR_GROUPS - 1)
            def _():
                start_in(i + _TR_NBUF, b)

            start_out(i, b)
        return carry

    lax.fori_loop(0, _TR_GROUPS, group, 0)

    for b in range(2):
        wait_out(b)
        start_in(120 + b, b)
    for b in range(2):
        wait_in(b)
        bouts[b][...] = bins[b][...].T
        start_out(120 + b, b)

    wait_out(2)
    tailT = lax.dot_general(
        tail_ref[...], eye_ref[...], (((0,), (0,)), ((), ())),
        preferred_element_type=jnp.float32,
    )
    bouts[2][pl.ds(0, _TR_TAIL), :] = tailT
    tcopy = pltpu.make_async_copy(
        bouts[2].at[pl.ds(0, _TR_TAIL), :],
        out_hbm.at[pl.ds(_TR_N * _TR_E, _TR_TAIL), :],
        stail,
    )
    tcopy.start()
    tcopy.wait()
    wait_out(3)
    for b in range(2):
        wait_out(b)


def _transpose(tableT, eye, tail, n_entities: int):
    scratch = (
        [pltpu.VMEM((_DIM, _TR_E), jnp.float32) for _ in range(_TR_NBUF)]
        + [pltpu.VMEM((_TR_E, _DIM), jnp.float32) for _ in range(_TR_NBUF)]
        + [pltpu.SemaphoreType.DMA for _ in range(2 * _TR_NBUF)]
        + [pltpu.SemaphoreType.DMA]
    )
    return pl.pallas_call(
        _stream_tr_body,
        in_specs=[
            pl.BlockSpec(memory_space=pltpu.VMEM),
            pl.BlockSpec(memory_space=pltpu.VMEM),
            pl.BlockSpec(memory_space=pl.ANY),
        ],
        out_specs=pl.BlockSpec(memory_space=pl.ANY),
        out_shape=jax.ShapeDtypeStruct((n_entities, _DIM), jnp.float32),
        scratch_shapes=scratch,
    )(eye, tail, tableT)


def _mlp_body(h_ref, t_ref, rel_ref, w1_ref, b1_ref, w2_ref, b2_ref, o_ref):
    r_avg = jnp.sum(rel_ref[...], axis=0, keepdims=True) * (1.0 / _NUM_REL)
    const = (
        jnp.dot(r_avg, w1_ref[_DIM : 2 * _DIM, :], preferred_element_type=jnp.float32)
        + b1_ref[...]
    )
    y = (
        jnp.dot(h_ref[...], w1_ref[0:_DIM, :], preferred_element_type=jnp.float32)
        + jnp.dot(t_ref[...], w1_ref[2 * _DIM : 3 * _DIM, :], preferred_element_type=jnp.float32)
        + const
    )
    y = y * 0.5 * (1.0 + lax.erf(y * np.float32(1.0 / np.sqrt(2.0))))
    zT = lax.dot_general(
        w2_ref[...], y, (((0,), (1,)), ((), ())),
        preferred_element_type=jnp.float32,
    )
    o_ref[...] = zT + b2_ref[...]


def _mlp(gathered, relp, W1, b1_2d, W2, b2_col, batch: int, block_b: int):
    grid = batch // block_b
    return pl.pallas_call(
        _mlp_body,
        grid=(grid,),
        in_specs=[
            pl.BlockSpec((block_b, _DIM), lambda i: (i, 0)),
            pl.BlockSpec((block_b, _DIM), lambda i, g=grid: (i + g, 0)),
            pl.BlockSpec((_REL_PAD, _DIM), lambda i: (0, 0)),
            pl.BlockSpec((3 * _DIM, _DIM), lambda i: (0, 0)),
            pl.BlockSpec((1, _DIM), lambda i: (0, 0)),
            pl.BlockSpec((_DIM, _NUM_REL), lambda i: (0, 0)),
            pl.BlockSpec((_NUM_REL, 1), lambda i: (0, 0)),
        ],
        out_specs=pl.BlockSpec((_NUM_REL, block_b), lambda i: (0, i)),
        out_shape=jax.ShapeDtypeStruct((_NUM_REL, batch), jnp.float32),
    )(gathered, gathered, relp, W1, b1_2d, W2, b2_col)


def kernel(src, tgt, entity_emb, relation_emb, W1, b1, W2, b2):
    batch = src.shape[0]
    n_entities = entity_emb.shape[0]
    idx = jnp.concatenate([src.astype(jnp.int32), tgt.astype(jnp.int32)])
    eye = jnp.eye(_DIM, dtype=jnp.float32)
    tableT = entity_emb.T
    tail = tableT[:, _TR_N * _TR_E :]
    table_rm = _transpose(tableT, eye, tail, n_entities)
    gathered = _gather_kernel(2 * batch, _DIM)(idx, table_rm)
    relp = jnp.zeros((_REL_PAD, _DIM), jnp.float32).at[:_NUM_REL].set(relation_emb)
    zT = _mlp(
        gathered,
        relp,
        W1,
        b1.reshape(1, _DIM),
        W2,
        b2.reshape(_NUM_REL, 1),
        batch,
        block_b=2048,
    )
    return zT.T


# speedup vs baseline: 1.0168x; 1.0168x over previous
import functools

import jax
import jax.numpy as jnp
import numpy as np
from jax import lax
from jax.experimental import pallas as pl
from jax.experimental.pallas import tpu as pltpu
from jax.experimental.pallas import tpu_sc as plsc

_DIM = 64
_NUM_REL = 500
_REL_PAD = 512

_NC = 2
_NS = 16
_NW = _NC * _NS
_GCHUNK = 128

_TE = 16384
_P_SPLIT = 30 * _TE
_P_TAIL = 576
_P_ROWS = _P_SPLIT + 31 * _TE + _P_TAIL - _P_SPLIT + _P_SPLIT
_P_ROWS = 31 * _TE + _P_TAIL
_GDT = jnp.bfloat16


@functools.lru_cache(maxsize=None)
def _gather_kernel(total_rows: int, dim: int):
    rows_per_w = total_rows // _NW
    n_chunks = rows_per_w // _GCHUNK
    mesh = plsc.VectorSubcoreMesh(core_axis_name="c", subcore_axis_name="s")

    @functools.partial(
        pl.kernel,
        mesh=mesh,
        out_type=jax.ShapeDtypeStruct((total_rows, dim), _GDT),
        scratch_types=[
            pltpu.VMEM((rows_per_w,), jnp.int32),
            pltpu.VMEM((rows_per_w, dim), _GDT),
            pltpu.SemaphoreType.DMA,
        ],
        compiler_params=pltpu.CompilerParams(use_tc_tiling_on_sc=False),
    )
    def gather(idx_hbm, table_hbm, out_hbm, idx_v, rows_v, sem):
        wid = lax.axis_index("s") * _NC + lax.axis_index("c")
        base = wid * rows_per_w
        pltpu.sync_copy(idx_hbm.at[pl.ds(base, rows_per_w)], idx_v)
        copies = [
            pltpu.async_copy(
                table_hbm.at[idx_v.at[pl.ds(j * _GCHUNK, _GCHUNK)]],
                rows_v.at[pl.ds(j * _GCHUNK, _GCHUNK), :],
                sem,
            )
            for j in range(n_chunks)
        ]
        for cp in copies:
            cp.wait()
        pltpu.sync_copy(rows_v, out_hbm.at[pl.ds(base, rows_per_w)])

    return gather


def _pairs_body(eye_ref, tail_ref, in_hbm, out_hbm, *rest):
    bl = rest[0:2]
    bh = rest[2:4]
    bo = rest[4:6]
    sl = rest[6:8]
    sh = rest[8:10]
    so = rest[10:12]
    stail = rest[12]

    def start_half(i, b, bufs, sems, off):
        for d in range(_DIM):
            pltpu.make_async_copy(
                in_hbm.at[d, pl.ds(off + i * _TE, _TE)],
                bufs[b].at[d],
                sems[b],
            ).start()

    def wait_half(b, bufs, sems):
        for d in range(_DIM):
            pltpu.make_async_copy(
                in_hbm.at[d, pl.ds(0, _TE)], bufs[b].at[d], sems[b]
            ).wait()

    def start_out(prow, b, width):
        pltpu.make_async_copy(
            bo[b].at[pl.ds(0, width), :],
            out_hbm.at[pl.ds(prow, width), :],
            so[b],
        ).start()

    def wait_out(b, width=_TE):
        pltpu.make_async_copy(
            bo[b].at[pl.ds(0, width), :],
            out_hbm.at[pl.ds(0, width), :],
            so[b],
        ).wait()

    for b in range(2):
        start_half(b, b, bl, sl, 0)
        start_half(b, b, bh, sh, _P_SPLIT)

    def group(j, carry):
        for b in range(2):
            i = j * 2 + b
            wait_half(b, bl, sl)
            wait_half(b, bh, sh)

            @pl.when(j > 0)
            def _():
                wait_out(b)

            bo[b][:, 0:_DIM] = bl[b][...].T.astype(_GDT)
            bo[b][:, _DIM : 2 * _DIM] = bh[b][...].T.astype(_GDT)

            @pl.when(j < 14)
            def _():
                start_half(i + 2, b, bl, sl, 0)
                start_half(i + 2, b, bh, sh, _P_SPLIT)

            start_out(i * _TE, b, _TE)
        return carry

    lax.fori_loop(0, 15, group, 0)

    wait_out(0)
    start_half(30, 0, bh, sh, _P_SPLIT)
    wait_half(0, bh, sh)
    bo[0][:, _DIM : 2 * _DIM] = bh[0][...].T.astype(_GDT)
    start_out(30 * _TE, 0, _TE)

    wait_out(1)
    tailT = lax.dot_general(
        tail_ref[...], eye_ref[...], (((0,), (0,)), ((), ())),
        preferred_element_type=jnp.float32,
    )
    bo[1][pl.ds(0, _P_TAIL), _DIM : 2 * _DIM] = tailT.astype(_GDT)
    tcopy = pltpu.make_async_copy(
        bo[1].at[pl.ds(0, _P_TAIL), :],
        out_hbm.at[pl.ds(31 * _TE, _P_TAIL), :],
        stail,
    )
    tcopy.start()
    tcopy.wait()
    wait_out(0)


def _pairs(tableT, eye, tail):
    scratch = (
        [pltpu.VMEM((_DIM, _TE), jnp.float32) for _ in range(4)]
        + [pltpu.VMEM((_TE, 2 * _DIM), _GDT) for _ in range(2)]
        + [pltpu.SemaphoreType.DMA for _ in range(6)]
        + [pltpu.SemaphoreType.DMA]
    )
    return pl.pallas_call(
        _pairs_body,
        in_specs=[
            pl.BlockSpec(memory_space=pltpu.VMEM),
            pl.BlockSpec(memory_space=pltpu.VMEM),
            pl.BlockSpec(memory_space=pl.ANY),
        ],
        out_specs=pl.BlockSpec(memory_space=pl.ANY),
        out_shape=jax.ShapeDtypeStruct((_P_ROWS, 2 * _DIM), _GDT),
        scratch_shapes=scratch,
    )(eye, tail, tableT)


def _mlp_body(h_ref, t_ref, sh_ref, st_ref, rel_ref, w1_ref, b1_ref, w2_ref,
              b2_ref, o_ref):
    hm = sh_ref[...] > 0.5
    tm = st_ref[...] > 0.5
    h = jnp.where(hm, h_ref[:, 0:_DIM], h_ref[:, _DIM : 2 * _DIM]).astype(
        jnp.float32
    )
    t = jnp.where(tm, t_ref[:, 0:_DIM], t_ref[:, _DIM : 2 * _DIM]).astype(
        jnp.float32
    )
    r_avg = jnp.sum(rel_ref[...], axis=0, keepdims=True) * (1.0 / _NUM_REL)
    const = (
        jnp.dot(r_avg, w1_ref[_DIM : 2 * _DIM, :], preferred_element_type=jnp.float32)
        + b1_ref[...]
    )
    y = (
        jnp.dot(h, w1_ref[0:_DIM, :], preferred_element_type=jnp.float32)
        + jnp.dot(t, w1_ref[2 * _DIM : 3 * _DIM, :], preferred_element_type=jnp.float32)
        + const
    )
    y = y * 0.5 * (1.0 + lax.erf(y * np.float32(1.0 / np.sqrt(2.0))))
    zT = lax.dot_general(
        w2_ref[...], y, (((0,), (1,)), ((), ())),
        preferred_element_type=jnp.float32,
    )
    o_ref[...] = zT + b2_ref[...]


def _mlp(gathered, sel, relp, W1, b1_2d, W2, b2_col, batch: int, block_b: int):
    grid = batch // block_b
    return pl.pallas_call(
        _mlp_body,
        grid=(grid,),
        in_specs=[
            pl.BlockSpec((block_b, 2 * _DIM), lambda i: (i, 0)),
            pl.BlockSpec((block_b, 2 * _DIM), lambda i, g=grid: (i + g, 0)),
            pl.BlockSpec((block_b, 1), lambda i: (i, 0)),
            pl.BlockSpec((block_b, 1), lambda i, g=grid: (i + g, 0)),
            pl.BlockSpec((_REL_PAD, _DIM), lambda i: (0, 0)),
            pl.BlockSpec((3 * _DIM, _DIM), lambda i: (0, 0)),
            pl.BlockSpec((1, _DIM), lambda i: (0, 0)),
            pl.BlockSpec((_DIM, _NUM_REL), lambda i: (0, 0)),
            pl.BlockSpec((_NUM_REL, 1), lambda i: (0, 0)),
        ],
        out_specs=pl.BlockSpec((_NUM_REL, block_b), lambda i: (0, i)),
        out_shape=jax.ShapeDtypeStruct((_NUM_REL, batch), jnp.float32),
    )(gathered, gathered, sel, sel, relp, W1, b1_2d, W2, b2_col)


def kernel(src, tgt, entity_emb, relation_emb, W1, b1, W2, b2):
    batch = src.shape[0]
    idx = jnp.concatenate([src.astype(jnp.int32), tgt.astype(jnp.int32)])
    sel = (idx < _P_SPLIT).astype(jnp.float32).reshape(2 * batch, 1)
    idx2 = jnp.where(idx < _P_SPLIT, idx, idx - _P_SPLIT)
    eye = jnp.eye(_DIM, dtype=jnp.float32)
    tableT = entity_emb.T
    tail = tableT[:, _P_SPLIT + 31 * _TE :]
    pair_table = _pairs(tableT, eye, tail)
    gathered = _gather_kernel(2 * batch, 2 * _DIM)(idx2, pair_table)
    relp = jnp.zeros((_REL_PAD, _DIM), jnp.float32).at[:_NUM_REL].set(relation_emb)
    zT = _mlp(
        gathered,
        sel,
        relp,
        W1,
        b1.reshape(1, _DIM),
        W2,
        b2.reshape(_NUM_REL, 1),
        batch,
        block_b=2048,
    )
    return zT.T


# speedup vs baseline: 2.4056x; 2.3658x over previous
import functools

import jax
import jax.numpy as jnp
import numpy as np
from jax import lax
from jax.experimental import pallas as pl
from jax.experimental.pallas import tpu as pltpu
from jax.experimental.pallas import tpu_sc as plsc

_DIM = 64
_NUM_REL = 500
_REL_PAD = 512

_NC = 2
_NS = 16
_NW = _NC * _NS
_GCHUNK = 128

_TE = 8192
_P_SPLIT = 60 * _TE
_P_TAIL = 576
_P_ROWS = 62 * _TE + _P_TAIL
_GDT = jnp.float32


@functools.lru_cache(maxsize=None)
def _gather_kernel(total_rows: int, dim: int):
    rows_per_w = total_rows // _NW
    n_chunks = rows_per_w // _GCHUNK
    mesh = plsc.VectorSubcoreMesh(core_axis_name="c", subcore_axis_name="s")

    @functools.partial(
        pl.kernel,
        mesh=mesh,
        out_type=jax.ShapeDtypeStruct((total_rows, dim), _GDT),
        scratch_types=[
            pltpu.VMEM((rows_per_w,), jnp.int32),
            pltpu.VMEM((rows_per_w // 2, dim), _GDT),
            pltpu.SemaphoreType.DMA,
        ],
        compiler_params=pltpu.CompilerParams(use_tc_tiling_on_sc=False),
    )
    def gather(idx_hbm, table_hbm, out_hbm, idx_v, rows_v, sem):
        wid = lax.axis_index("s") * _NC + lax.axis_index("c")
        base = wid * rows_per_w
        pltpu.sync_copy(idx_hbm.at[pl.ds(base, rows_per_w)], idx_v)
        half = rows_per_w // 2
        for hhalf in range(2):
            copies = [
                pltpu.async_copy(
                    table_hbm.at[
                        idx_v.at[pl.ds(hhalf * half + j * _GCHUNK, _GCHUNK)]
                    ],
                    rows_v.at[pl.ds(j * _GCHUNK, _GCHUNK), :],
                    sem,
                )
                for j in range(n_chunks // 2)
            ]
            for cp in copies:
                cp.wait()
            pltpu.sync_copy(
                rows_v, out_hbm.at[pl.ds(base + hhalf * half, half)]
            )

    return gather


def _pairs_body(eye_ref, tail_ref, in_hbm, out_hbm, *rest):
    bl = rest[0:2]
    bh = rest[2:4]
    bo = rest[4:6]
    sl = rest[6:8]
    sh = rest[8:10]
    so = rest[10:12]
    stail = rest[12]

    def start_half(i, b, bufs, sems, off):
        for d in range(_DIM):
            pltpu.make_async_copy(
                in_hbm.at[d, pl.ds(off + i * _TE, _TE)],
                bufs[b].at[d],
                sems[b],
            ).start()

    def wait_half(b, bufs, sems):
        for d in range(_DIM):
            pltpu.make_async_copy(
                in_hbm.at[d, pl.ds(0, _TE)], bufs[b].at[d], sems[b]
            ).wait()

    def start_out(prow, b, width):
        pltpu.make_async_copy(
            bo[b].at[pl.ds(0, width), :],
            out_hbm.at[pl.ds(prow, width), :],
            so[b],
        ).start()

    def wait_out(b, width=_TE):
        pltpu.make_async_copy(
            bo[b].at[pl.ds(0, width), :],
            out_hbm.at[pl.ds(0, width), :],
            so[b],
        ).wait()

    for b in range(2):
        start_half(b, b, bl, sl, 0)
        start_half(b, b, bh, sh, _P_SPLIT)

    def group(j, carry):
        for b in range(2):
            i = j * 2 + b
            wait_half(b, bl, sl)
            wait_half(b, bh, sh)

            @pl.when(j > 0)
            def _():
                wait_out(b)

            bo[b][:, 0:_DIM] = bl[b][...].T.astype(_GDT)
            bo[b][:, _DIM : 2 * _DIM] = bh[b][...].T.astype(_GDT)

            @pl.when(j < 29)
            def _():
                start_half(i + 2, b, bl, sl, 0)
                start_half(i + 2, b, bh, sh, _P_SPLIT)

            start_out(i * _TE, b, _TE)
        return carry

    lax.fori_loop(0, 30, group, 0)

    for b in range(2):
        wait_out(b)
        start_half(60 + b, b, bh, sh, _P_SPLIT)
    for b in range(2):
        wait_half(b, bh, sh)
        bo[b][:, _DIM : 2 * _DIM] = bh[b][...].T.astype(_GDT)
        start_out((60 + b) * _TE, b, _TE)

    wait_out(0)
    tailT = lax.dot_general(
        tail_ref[...], eye_ref[...], (((0,), (0,)), ((), ())),
        preferred_element_type=jnp.float32,
    )
    bo[0][pl.ds(0, _P_TAIL), _DIM : 2 * _DIM] = tailT.astype(_GDT)
    tcopy = pltpu.make_async_copy(
        bo[0].at[pl.ds(0, _P_TAIL), :],
        out_hbm.at[pl.ds(62 * _TE, _P_TAIL), :],
        stail,
    )
    tcopy.start()
    tcopy.wait()
    wait_out(1)


def _pairs(tableT, eye, tail):
    scratch = (
        [pltpu.VMEM((_DIM, _TE), jnp.float32) for _ in range(4)]
        + [pltpu.VMEM((_TE, 2 * _DIM), _GDT) for _ in range(2)]
        + [pltpu.SemaphoreType.DMA for _ in range(6)]
        + [pltpu.SemaphoreType.DMA]
    )
    return pl.pallas_call(
        _pairs_body,
        in_specs=[
            pl.BlockSpec(memory_space=pltpu.VMEM),
            pl.BlockSpec(memory_space=pltpu.VMEM),
            pl.BlockSpec(memory_space=pl.ANY),
        ],
        out_specs=pl.BlockSpec(memory_space=pl.ANY),
        out_shape=jax.ShapeDtypeStruct((_P_ROWS, 2 * _DIM), _GDT),
        scratch_shapes=scratch,
    )(eye, tail, tableT)


def _mlp_body(h_ref, t_ref, sh_ref, st_ref, rel_ref, w1_ref, b1_ref, w2_ref,
              b2_ref, o_ref):
    hm = sh_ref[...] > 0.5
    tm = st_ref[...] > 0.5
    h = jnp.where(hm, h_ref[:, 0:_DIM], h_ref[:, _DIM : 2 * _DIM]).astype(
        jnp.float32
    )
    t = jnp.where(tm, t_ref[:, 0:_DIM], t_ref[:, _DIM : 2 * _DIM]).astype(
        jnp.float32
    )
    r_avg = jnp.sum(rel_ref[...], axis=0, keepdims=True) * (1.0 / _NUM_REL)
    const = (
        jnp.dot(r_avg, w1_ref[_DIM : 2 * _DIM, :], preferred_element_type=jnp.float32)
        + b1_ref[...]
    )
    y = (
        jnp.dot(h, w1_ref[0:_DIM, :], preferred_element_type=jnp.float32)
        + jnp.dot(t, w1_ref[2 * _DIM : 3 * _DIM, :], preferred_element_type=jnp.float32)
        + const
    )
    y = y * 0.5 * (1.0 + lax.erf(y * np.float32(1.0 / np.sqrt(2.0))))
    zT = lax.dot_general(
        w2_ref[...], y, (((0,), (1,)), ((), ())),
        preferred_element_type=jnp.float32,
    )
    o_ref[...] = zT + b2_ref[...]


def _mlp(gathered, sel, relp, W1, b1_2d, W2, b2_col, batch: int, block_b: int):
    grid = batch // block_b
    return pl.pallas_call(
        _mlp_body,
        grid=(grid,),
        in_specs=[
            pl.BlockSpec((block_b, 2 * _DIM), lambda i: (i, 0)),
            pl.BlockSpec((block_b, 2 * _DIM), lambda i, g=grid: (i + g, 0)),
            pl.BlockSpec((block_b, 1), lambda i: (i, 0)),
            pl.BlockSpec((block_b, 1), lambda i, g=grid: (i + g, 0)),
            pl.BlockSpec((_REL_PAD, _DIM), lambda i: (0, 0)),
            pl.BlockSpec((3 * _DIM, _DIM), lambda i: (0, 0)),
            pl.BlockSpec((1, _DIM), lambda i: (0, 0)),
            pl.BlockSpec((_DIM, _NUM_REL), lambda i: (0, 0)),
            pl.BlockSpec((_NUM_REL, 1), lambda i: (0, 0)),
        ],
        out_specs=pl.BlockSpec((_NUM_REL, block_b), lambda i: (0, i)),
        out_shape=jax.ShapeDtypeStruct((_NUM_REL, batch), jnp.float32),
    )(gathered, gathered, sel, sel, relp, W1, b1_2d, W2, b2_col)


def kernel(src, tgt, entity_emb, relation_emb, W1, b1, W2, b2):
    batch = src.shape[0]
    idx = jnp.concatenate([src.astype(jnp.int32), tgt.astype(jnp.int32)])
    sel = (idx < _P_SPLIT).astype(jnp.float32).reshape(2 * batch, 1)
    idx2 = jnp.where(idx < _P_SPLIT, idx, idx - _P_SPLIT)
    eye = jnp.eye(_DIM, dtype=jnp.float32)
    tableT = entity_emb.T
    tail = tableT[:, _P_SPLIT + 62 * _TE :]
    pair_table = _pairs(tableT, eye, tail)
    gathered = _gather_kernel(2 * batch, 2 * _DIM)(idx2, pair_table)
    relp = jnp.zeros((_REL_PAD, _DIM), jnp.float32).at[:_NUM_REL].set(relation_emb)
    zT = _mlp(
        gathered,
        sel,
        relp,
        W1,
        b1.reshape(1, _DIM),
        W2,
        b2.reshape(_NUM_REL, 1),
        batch,
        block_b=2048,
    )
    return zT.T
